# Initial kernel scaffold; baseline (speedup 1.0000x reference)
import jax
import jax.numpy as jnp
from jax.experimental import pallas as pl


def kernel(h, norm, prev_h, loop_weight, skip_connect_weight, skip_connect_bias, edge_index):
    raise NotImplementedError("write your pallas kernel here")


# speedup vs baseline: 7.4261x; 7.4261x over previous
import functools

import jax
import jax.numpy as jnp
from jax import lax
from jax.experimental import pallas as pl
from jax.experimental.pallas import tpu as pltpu
from jax.experimental.pallas import tpu_sc as plsc

_N = 10000
_E = 320000
_F = 128

_NC = 2
_NS = 16
_NW = _NC * _NS
_EPW = _E // _NW
_K = 80
_CHUNKS = _EPW // _K
_WT = 10
_RPT = _N // _WT


def _sc_scatter_body(src_hbm, dst_hbm, h_hbm, out_hbm,
                     src_v, dst_v, rows_v, agg_sh, sem):
    cid = lax.axis_index("c")
    sid = lax.axis_index("s")
    wid = cid * _NS + sid

    @pl.when(sid < _WT)
    def _zero():
        def _zrow(i, carry):
            for cbase in range(_F // 16):
                rows_v[i, pl.ds(cbase * 16, 16)] = jnp.zeros((16,), jnp.float32)
            return carry
        lax.fori_loop(0, _K, _zrow, 0)
        for r in range(_RPT // _K):
            pltpu.sync_copy(rows_v,
                            agg_sh.at[pl.ds(sid * _RPT + r * _K, _K)])
        pltpu.sync_copy(rows_v.at[pl.ds(0, _RPT % _K)],
                        agg_sh.at[pl.ds(sid * _RPT + (_RPT // _K) * _K,
                                        _RPT % _K)])
    plsc.subcore_barrier()

    pltpu.sync_copy(src_hbm.at[wid], src_v)
    pltpu.sync_copy(dst_hbm.at[wid], dst_v)

    def _chunk(j, carry):
        pltpu.async_copy(h_hbm.at[src_v.at[j]], rows_v, sem).wait()
        pltpu.sync_copy(rows_v, agg_sh.at[dst_v.at[j]], add=True)
        return carry
    lax.fori_loop(0, _CHUNKS, _chunk, 0)
    plsc.subcore_barrier()

    @pl.when(sid < _WT)
    def _writeout():
        pltpu.sync_copy(agg_sh.at[pl.ds(sid * _RPT, _RPT)],
                        out_hbm.at[cid, pl.ds(sid * _RPT, _RPT)])


_sc_scatter = functools.partial(
    pl.kernel,
    out_type=jax.ShapeDtypeStruct((_NC, _N, _F), jnp.float32),
    mesh=plsc.VectorSubcoreMesh(core_axis_name="c", subcore_axis_name="s",
                                num_cores=_NC, num_subcores=_NS),
    scratch_types=[
        pltpu.VMEM((_CHUNKS, _K), jnp.int32),
        pltpu.VMEM((_CHUNKS, _K), jnp.int32),
        pltpu.VMEM((_K, _F), jnp.float32),
        pltpu.VMEM_SHARED((_N, _F), jnp.float32),
        pltpu.SemaphoreType.DMA,
    ],
)(_sc_scatter_body)


def _tc_epilogue_body(h_ref, prev_ref, norm_ref, agg0_ref, agg1_ref,
                      wl_ref, wsk_ref, b_ref, out_ref):
    prev = prev_ref[...]
    sw = jax.nn.sigmoid(
        jnp.dot(prev, wsk_ref[...], preferred_element_type=jnp.float32)
        + b_ref[...])
    lm = jnp.dot(h_ref[...], wl_ref[...], preferred_element_type=jnp.float32)
    node = (agg0_ref[...] + agg1_ref[...]) * norm_ref[...] + lm
    out = sw * node + (1.0 - sw) * prev
    u = lax.bitcast_convert_type(out, jnp.uint32)
    lsb = (u >> 13) & jnp.uint32(1)
    u = (u + jnp.uint32(0x0FFF) + lsb) & jnp.uint32(0xFFFFE000)
    out_ref[...] = lax.bitcast_convert_type(u, jnp.float32)


def _tc_epilogue(h, prev_h, norm, agg0, agg1, wl, wsk, b):
    blk = 1000
    grid = (_N // blk,)
    row_spec = pl.BlockSpec((blk, _F), lambda i: (i, 0))
    full_spec = pl.BlockSpec((_F, _F), lambda i: (0, 0))
    return pl.pallas_call(
        _tc_epilogue_body,
        grid=grid,
        in_specs=[
            row_spec,
            row_spec,
            pl.BlockSpec((blk, 1), lambda i: (i, 0)),
            row_spec,
            row_spec,
            full_spec,
            full_spec,
            pl.BlockSpec((1, _F), lambda i: (0, 0)),
        ],
        out_specs=row_spec,
        out_shape=jax.ShapeDtypeStruct((_N, _F), jnp.float32),
    )(h, prev_h, norm, agg0, agg1, wl, wsk, b)


def kernel(h, norm, prev_h, loop_weight, skip_connect_weight,
           skip_connect_bias, edge_index):
    src = edge_index[0].reshape(_NW, _CHUNKS, _K)
    dst = edge_index[1].reshape(_NW, _CHUNKS, _K)
    parts = _sc_scatter(src, dst, h)
    return _tc_epilogue(h, prev_h, norm, parts[0], parts[1],
                        loop_weight, skip_connect_weight,
                        skip_connect_bias.reshape(1, _F))


# speedup vs baseline: 9.2216x; 1.2418x over previous
import functools

import jax
import jax.numpy as jnp
from jax import lax
from jax.experimental import pallas as pl
from jax.experimental.pallas import tpu as pltpu
from jax.experimental.pallas import tpu_sc as plsc

_N = 10000
_E = 320000
_F = 128

_NC = 2
_NS = 16
_NW = _NC * _NS
_EPW = _E // _NW
_K = 80
_CHUNKS = _EPW // _K
_PAIRS = _CHUNKS // 2
_WT = 10
_RPT = _N // _WT
_ZR = 40


def _sc_scatter_body(src_hbm, dst_hbm, h_hbm, out_hbm,
                     src_v, dst_v, rows0_v, rows1_v, agg_sh,
                     gsem, ssem0, ssem1):
    cid = lax.axis_index("c")
    sid = lax.axis_index("s")
    wid = cid * _NS + sid

    @pl.when(sid < _WT)
    def _zero():
        def _zrow(i, carry):
            for cbase in range(_F // 16):
                rows0_v[i, pl.ds(cbase * 16, 16)] = jnp.zeros((16,),
                                                              jnp.float32)
            return carry
        lax.fori_loop(0, _ZR, _zrow, 0)
        for r in range(_RPT // _ZR):
            pltpu.sync_copy(rows0_v.at[pl.ds(0, _ZR)],
                            agg_sh.at[pl.ds(sid * _RPT + r * _ZR, _ZR)])
    plsc.subcore_barrier()

    pltpu.sync_copy(src_hbm.at[wid], src_v)
    pltpu.sync_copy(dst_hbm.at[wid], dst_v)

    def _gather(j, buf):
        pltpu.async_copy(h_hbm.at[src_v.at[pl.ds(j * _K, _K)]], buf, gsem)

    def _gather_wait(buf):
        pltpu.make_async_copy(h_hbm.at[src_v.at[pl.ds(0, _K)]], buf,
                              gsem).wait()

    def _scatter(j, buf, sem):
        pltpu.async_copy(buf, agg_sh.at[dst_v.at[j]], sem, add=True)

    def _scatter_wait(buf, sem):
        pltpu.make_async_copy(buf, agg_sh.at[dst_v.at[0]], sem).wait()

    _gather(0, rows0_v)

    def _pair(p, carry):
        j0 = 2 * p
        j1 = j0 + 1
        _gather_wait(rows0_v)
        _scatter(j0, rows0_v, ssem0)

        @pl.when(p > 0)
        def _():
            _scatter_wait(rows1_v, ssem1)
        _gather(j1, rows1_v)
        _gather_wait(rows1_v)
        _scatter_wait(rows0_v, ssem0)
        _scatter(j1, rows1_v, ssem1)
        _gather(j0 + 2, rows0_v)
        return carry

    lax.fori_loop(0, _PAIRS, _pair, 0)
    _gather_wait(rows0_v)
    _scatter_wait(rows1_v, ssem1)
    _scatter(_CHUNKS - 1, rows0_v, ssem0)
    _scatter_wait(rows0_v, ssem0)
    plsc.subcore_barrier()

    @pl.when(sid < _WT)
    def _writeout():
        pltpu.sync_copy(agg_sh.at[pl.ds(sid * _RPT, _RPT)],
                        out_hbm.at[cid, pl.ds(sid * _RPT, _RPT)])


_sc_scatter = functools.partial(
    pl.kernel,
    out_type=jax.ShapeDtypeStruct((_NC, _N, _F), jnp.float32),
    mesh=plsc.VectorSubcoreMesh(core_axis_name="c", subcore_axis_name="s",
                                num_cores=_NC, num_subcores=_NS),
    scratch_types=[
        pltpu.VMEM((_EPW,), jnp.int32),
        pltpu.VMEM((_CHUNKS, _K), jnp.int32),
        pltpu.VMEM((_K, _F), jnp.float32),
        pltpu.VMEM((_K, _F), jnp.float32),
        pltpu.VMEM_SHARED((_N, _F), jnp.float32),
        pltpu.SemaphoreType.DMA,
        pltpu.SemaphoreType.DMA,
        pltpu.SemaphoreType.DMA,
    ],
)(_sc_scatter_body)


def _tc_epilogue_body(h_ref, prev_ref, norm_ref, agg0_ref, agg1_ref,
                      wl_ref, wsk_ref, b_ref, out_ref):
    prev = prev_ref[...]
    sw = jax.nn.sigmoid(
        jnp.dot(prev, wsk_ref[...], preferred_element_type=jnp.float32)
        + b_ref[...])
    lm = jnp.dot(h_ref[...], wl_ref[...], preferred_element_type=jnp.float32)
    node = (agg0_ref[...] + agg1_ref[...]) * norm_ref[...] + lm
    out = sw * node + (1.0 - sw) * prev
    u = lax.bitcast_convert_type(out, jnp.uint32)
    lsb = (u >> 13) & jnp.uint32(1)
    u = (u + jnp.uint32(0x0FFF) + lsb) & jnp.uint32(0xFFFFE000)
    out_ref[...] = lax.bitcast_convert_type(u, jnp.float32)


def _tc_epilogue(h, prev_h, norm, agg0, agg1, wl, wsk, b):
    blk = 1000
    grid = (_N // blk,)
    row_spec = pl.BlockSpec((blk, _F), lambda i: (i, 0))
    full_spec = pl.BlockSpec((_F, _F), lambda i: (0, 0))
    return pl.pallas_call(
        _tc_epilogue_body,
        grid=grid,
        in_specs=[
            row_spec,
            row_spec,
            pl.BlockSpec((blk, 1), lambda i: (i, 0)),
            row_spec,
            row_spec,
            full_spec,
            full_spec,
            pl.BlockSpec((1, _F), lambda i: (0, 0)),
        ],
        out_specs=row_spec,
        out_shape=jax.ShapeDtypeStruct((_N, _F), jnp.float32),
    )(h, prev_h, norm, agg0, agg1, wl, wsk, b)


def kernel(h, norm, prev_h, loop_weight, skip_connect_weight,
           skip_connect_bias, edge_index):
    src = edge_index[0].reshape(_NW, _EPW)
    dst = edge_index[1].reshape(_NW, _CHUNKS, _K)
    parts = _sc_scatter(src, dst, h)
    return _tc_epilogue(h, prev_h, norm, parts[0], parts[1],
                        loop_weight, skip_connect_weight,
                        skip_connect_bias.reshape(1, _F))
